# Initial kernel scaffold; baseline (speedup 1.0000x reference)
import jax
import jax.numpy as jnp
from jax.experimental import pallas as pl


def kernel(hidden_states, gate_w, w1, w3, w2):
    raise NotImplementedError("write your pallas kernel here")


# speedup vs baseline: 1.4855x; 1.4855x over previous
import functools

import jax
import jax.numpy as jnp
from jax.experimental import pallas as pl
from jax.experimental.pallas import tpu as pltpu

E = 8
TOP_K = 2
H = 1024
FF = 512
TBLK = 256


def _moe_block_kernel(x_ref, gate_ref, w1_ref, w3_ref, w2_ref,
                      out_ref, logits_ref, acc_ref, combine_ref):
    e = pl.program_id(1)
    x = x_ref[...]

    @pl.when(e == 0)
    def _router():
        logits = jax.lax.dot_general(
            x, gate_ref[...], (((1,), (1,)), ((), ())),
            preferred_element_type=jnp.float32)
        logits_ref[...] = logits
        scores = jax.nn.sigmoid(logits)
        i1 = jnp.argmax(scores, axis=1)
        v1 = jnp.max(scores, axis=1)
        cols = jax.lax.broadcasted_iota(jnp.int32, scores.shape, 1)
        masked = jnp.where(cols == i1[:, None], -jnp.inf, scores)
        i2 = jnp.argmax(masked, axis=1)
        v2 = jnp.max(masked, axis=1)
        denom = v1 + v2 + 1e-6
        w1n = (v1 / denom)[:, None]
        w2n = (v2 / denom)[:, None]
        combine_ref[...] = (jnp.where(cols == i1[:, None], w1n, 0.0)
                            + jnp.where(cols == i2[:, None], w2n, 0.0))
        acc_ref[...] = jnp.zeros_like(acc_ref)

    w1e = w1_ref[0]
    w3e = w3_ref[0]
    w2e = w2_ref[0]
    h1 = jax.lax.dot_general(x, w1e, (((1,), (1,)), ((), ())),
                             preferred_element_type=jnp.float32)
    h3 = jax.lax.dot_general(x, w3e, (((1,), (1,)), ((), ())),
                             preferred_element_type=jnp.float32)
    he = (h1 * jax.nn.sigmoid(h1)) * h3
    ye = jax.lax.dot_general(he, w2e, (((1,), (1,)), ((), ())),
                             preferred_element_type=jnp.float32)
    comb = combine_ref[...]
    ecols = jax.lax.broadcasted_iota(jnp.int32, comb.shape, 1)
    wcol = jnp.sum(jnp.where(ecols == e, comb, 0.0), axis=1, keepdims=True)
    acc_ref[...] += wcol * ye

    @pl.when(e == E - 1)
    def _store():
        out_ref[...] = acc_ref[...]


def kernel(hidden_states, gate_w, w1, w3, w2):
    b, s, h = hidden_states.shape
    x = hidden_states.reshape(-1, h)
    T = x.shape[0]
    nblk = T // TBLK

    grid = (nblk, E)
    out, logits = pl.pallas_call(
        _moe_block_kernel,
        grid=grid,
        in_specs=[
            pl.BlockSpec((TBLK, H), lambda i, e: (i, 0)),
            pl.BlockSpec((E, H), lambda i, e: (0, 0)),
            pl.BlockSpec((1, FF, H), lambda i, e: (e, 0, 0)),
            pl.BlockSpec((1, FF, H), lambda i, e: (e, 0, 0)),
            pl.BlockSpec((1, H, FF), lambda i, e: (e, 0, 0)),
        ],
        out_specs=[
            pl.BlockSpec((TBLK, H), lambda i, e: (i, 0)),
            pl.BlockSpec((TBLK, E), lambda i, e: (i, 0)),
        ],
        out_shape=[
            jax.ShapeDtypeStruct((T, H), jnp.float32),
            jax.ShapeDtypeStruct((T, E), jnp.float32),
        ],
        scratch_shapes=[
            pltpu.VMEM((TBLK, H), jnp.float32),
            pltpu.VMEM((TBLK, E), jnp.float32),
        ],
    )(x, gate_w, w1, w3, w2)

    return out.reshape(b, s, h), logits


# speedup vs baseline: 3.0141x; 2.0290x over previous
import functools

import jax
import jax.numpy as jnp
from jax.experimental import pallas as pl
from jax.experimental.pallas import tpu as pltpu

E = 8
TOP_K = 2
H = 1024
FF = 512
TBLK = 1024


def _moe_block_kernel(x_ref, gate_ref, w1_ref, w3_ref, w2_ref,
                      out_ref, logits_ref, acc_ref, combine_ref):
    e = pl.program_id(1)
    x = x_ref[...]

    @pl.when(e == 0)
    def _router():
        logits = jax.lax.dot_general(
            x, gate_ref[...], (((1,), (1,)), ((), ())),
            preferred_element_type=jnp.float32)
        logits_ref[...] = logits
        scores = jax.nn.sigmoid(logits)
        i1 = jnp.argmax(scores, axis=1)
        v1 = jnp.max(scores, axis=1)
        cols = jax.lax.broadcasted_iota(jnp.int32, scores.shape, 1)
        masked = jnp.where(cols == i1[:, None], -jnp.inf, scores)
        i2 = jnp.argmax(masked, axis=1)
        v2 = jnp.max(masked, axis=1)
        denom = v1 + v2 + 1e-6
        w1n = (v1 / denom)[:, None]
        w2n = (v2 / denom)[:, None]
        combine_ref[...] = (jnp.where(cols == i1[:, None], w1n, 0.0)
                            + jnp.where(cols == i2[:, None], w2n, 0.0))
        acc_ref[...] = jnp.zeros_like(acc_ref)

    w1e = w1_ref[0]
    w3e = w3_ref[0]
    w2e = w2_ref[0]
    h1 = jax.lax.dot_general(x, w1e, (((1,), (1,)), ((), ())),
                             preferred_element_type=jnp.float32)
    h3 = jax.lax.dot_general(x, w3e, (((1,), (1,)), ((), ())),
                             preferred_element_type=jnp.float32)
    he = (h1 * jax.nn.sigmoid(h1)) * h3
    ye = jax.lax.dot_general(he, w2e, (((1,), (1,)), ((), ())),
                             preferred_element_type=jnp.float32)
    comb = combine_ref[...]
    ecols = jax.lax.broadcasted_iota(jnp.int32, comb.shape, 1)
    wcol = jnp.sum(jnp.where(ecols == e, comb, 0.0), axis=1, keepdims=True)
    acc_ref[...] += wcol * ye

    @pl.when(e == E - 1)
    def _store():
        out_ref[...] = acc_ref[...]


def kernel(hidden_states, gate_w, w1, w3, w2):
    b, s, h = hidden_states.shape
    x = hidden_states.reshape(-1, h)
    T = x.shape[0]
    nblk = T // TBLK

    grid = (nblk, E)
    out, logits = pl.pallas_call(
        _moe_block_kernel,
        grid=grid,
        in_specs=[
            pl.BlockSpec((TBLK, H), lambda i, e: (i, 0)),
            pl.BlockSpec((E, H), lambda i, e: (0, 0)),
            pl.BlockSpec((1, FF, H), lambda i, e: (e, 0, 0)),
            pl.BlockSpec((1, FF, H), lambda i, e: (e, 0, 0)),
            pl.BlockSpec((1, H, FF), lambda i, e: (e, 0, 0)),
        ],
        out_specs=[
            pl.BlockSpec((TBLK, H), lambda i, e: (i, 0)),
            pl.BlockSpec((TBLK, E), lambda i, e: (i, 0)),
        ],
        out_shape=[
            jax.ShapeDtypeStruct((T, H), jnp.float32),
            jax.ShapeDtypeStruct((T, E), jnp.float32),
        ],
        scratch_shapes=[
            pltpu.VMEM((TBLK, H), jnp.float32),
            pltpu.VMEM((TBLK, E), jnp.float32),
        ],
    )(x, gate_w, w1, w3, w2)

    return out.reshape(b, s, h), logits
